# baseline (speedup 1.0000x reference)
import functools
import jax
import jax.numpy as jnp
from jax.experimental import pallas as pl
from jax.experimental.pallas import tpu as pltpu

_N = 10000
_E = 160000
_R = 3
_NPAD = 10240
_OTILE = 2048
_NTILE = 2048
_EBLK = 512
_EPAD = 160256
_EB = _EPAD // _EBLK
_OT = (_R * _NPAD) // _OTILE
_NT = _NPAD // _NTILE
_BIG = jnp.int32(1 << 30)


def _gather_body(src_ref, x_ref, msg_ref):
    n = pl.program_id(2)

    @pl.when(n == 0)
    def _():
        msg_ref[...] = jnp.zeros_like(msg_ref)

    s = src_ref[...]
    col = jax.lax.broadcasted_iota(jnp.int32, (_EBLK, _NTILE), 1) + n * _NTILE
    oh = (s == col).astype(jnp.float32)
    msg_ref[...] += jnp.dot(oh, x_ref[...], preferred_element_type=jnp.float32)


def _scatter_body(idx_ref, msg_ref, s_ref, cnt_ref):
    o = pl.program_id(0)
    e = pl.program_id(2)

    @pl.when(e == 0)
    def _():
        s_ref[...] = jnp.zeros_like(s_ref)
        cnt_ref[...] = jnp.zeros_like(cnt_ref)

    ids = idx_ref[pl.ds(e, 1), :]
    row = jax.lax.broadcasted_iota(jnp.int32, (_OTILE, _EBLK), 0) + o * _OTILE
    oh = (ids == row).astype(jnp.float32)
    s_ref[...] += jnp.dot(oh, msg_ref[...], preferred_element_type=jnp.float32)
    cnt_ref[...] += jnp.sum(oh, axis=1, keepdims=True)

    @pl.when(e == _EB - 1)
    def _():
        s_ref[...] = s_ref[...] / jnp.maximum(cnt_ref[...], 1.0)


def _seg_mean(h, src2d, idxrow, d):
    dblk = min(d, 512)
    dt = d // dblk
    msg = pl.pallas_call(
        _gather_body,
        grid=(_EB, dt, _NT),
        in_specs=[
            pl.BlockSpec((_EBLK, 1), lambda e, dd, n: (e, 0)),
            pl.BlockSpec((_NTILE, dblk), lambda e, dd, n: (n, dd)),
        ],
        out_specs=pl.BlockSpec((_EBLK, dblk), lambda e, dd, n: (e, dd)),
        out_shape=jax.ShapeDtypeStruct((_EPAD, d), jnp.float32),
    )(src2d, h)
    s = pl.pallas_call(
        _scatter_body,
        grid=(_OT, dt, _EB),
        in_specs=[
            pl.BlockSpec((_EB, _EBLK), lambda o, dd, e: (0, 0)),
            pl.BlockSpec((_EBLK, dblk), lambda o, dd, e: (e, dd)),
        ],
        out_specs=pl.BlockSpec((_OTILE, dblk), lambda o, dd, e: (o, dd)),
        out_shape=jax.ShapeDtypeStruct((_R * _NPAD, d), jnp.float32),
        scratch_shapes=[pltpu.VMEM((_OTILE, 1), jnp.float32)],
    )(idxrow, msg)
    return s


def _dense_body(relu, x_ref, s0_ref, s1_ref, s2_ref, wr_ref, w0_ref, w1_ref,
                w2_ref, b_ref, out_ref):
    acc = jnp.dot(x_ref[...], wr_ref[...], preferred_element_type=jnp.float32)
    acc += jnp.dot(s0_ref[...], w0_ref[0], preferred_element_type=jnp.float32)
    acc += jnp.dot(s1_ref[...], w1_ref[0], preferred_element_type=jnp.float32)
    acc += jnp.dot(s2_ref[...], w2_ref[0], preferred_element_type=jnp.float32)
    acc += b_ref[...]
    if relu:
        acc = jnp.maximum(acc, 0.0)
    out_ref[...] = acc


def _dense(h, s, w_rel, w_root, b, relu):
    din = h.shape[1]
    dout = w_root.shape[1]
    bn = 512
    bdo = min(dout, 256)
    npt = _NPAD // bn
    body = functools.partial(_dense_body, relu)
    return pl.pallas_call(
        body,
        grid=(_NPAD // bn, dout // bdo),
        in_specs=[
            pl.BlockSpec((bn, din), lambda i, j: (i, 0)),
            pl.BlockSpec((bn, din), lambda i, j: (i, 0)),
            pl.BlockSpec((bn, din), lambda i, j: (i + npt, 0)),
            pl.BlockSpec((bn, din), lambda i, j: (i + 2 * npt, 0)),
            pl.BlockSpec((din, bdo), lambda i, j: (0, j)),
            pl.BlockSpec((1, din, bdo), lambda i, j: (0, 0, j)),
            pl.BlockSpec((1, din, bdo), lambda i, j: (1, 0, j)),
            pl.BlockSpec((1, din, bdo), lambda i, j: (2, 0, j)),
            pl.BlockSpec((1, bdo), lambda i, j: (0, j)),
        ],
        out_specs=pl.BlockSpec((bn, bdo), lambda i, j: (i, j)),
        out_shape=jax.ShapeDtypeStruct((_NPAD, dout), jnp.float32),
    )(h, s, s, s, w_root, w_rel, w_rel, w_rel, b.reshape(1, dout))


def _head_body(h_ref, w_ref, b_ref, out_ref):
    logits = jnp.dot(h_ref[...], w_ref[...], preferred_element_type=jnp.float32)
    logits += b_ref[...]
    col = jax.lax.broadcasted_iota(jnp.int32, logits.shape, 1)
    logits = jnp.where(col < 40, logits, -1e30)
    m = jnp.max(logits, axis=1, keepdims=True)
    lse = jnp.log(jnp.sum(jnp.exp(logits - m), axis=1, keepdims=True)) + m
    out_ref[...] = logits - lse


def _head(h, fc_w, fc_b):
    wpad = jnp.zeros((128, 128), jnp.float32).at[:, :40].set(fc_w)
    bpad = jnp.zeros((1, 128), jnp.float32).at[0, :40].set(fc_b)
    return pl.pallas_call(
        _head_body,
        grid=(_NPAD // 512,),
        in_specs=[
            pl.BlockSpec((512, 128), lambda i: (i, 0)),
            pl.BlockSpec((128, 128), lambda i: (0, 0)),
            pl.BlockSpec((1, 128), lambda i: (0, 0)),
        ],
        out_specs=pl.BlockSpec((512, 128), lambda i: (i, 0)),
        out_shape=jax.ShapeDtypeStruct((_NPAD, 128), jnp.float32),
    )(h, wpad, bpad)


@jax.jit
def kernel(x, edge_index, edge_type, W1_rel, W1_root, b1, W2_rel, W2_root, b2,
           W3_rel, W3_root, b3, fc_w, fc_b):
    src = edge_index[0]
    dst = edge_index[1]
    idx = edge_type * _NPAD + dst
    pad = _EPAD - _E
    src_p = jnp.concatenate([src, jnp.full((pad,), _BIG, jnp.int32)])
    idx_p = jnp.concatenate([idx, jnp.full((pad,), _BIG, jnp.int32)])
    src2d = src_p.reshape(_EPAD, 1)
    idxrow = idx_p.reshape(_EB, _EBLK)

    h = jnp.zeros((_NPAD, 128), jnp.float32).at[:_N].set(x)
    for w_rel, w_root, b, relu in (
        (W1_rel, W1_root, b1, True),
        (W2_rel, W2_root, b2, True),
        (W3_rel, W3_root, b3, True),
    ):
        s = _seg_mean(h, src2d, idxrow, h.shape[1])
        h = _dense(h, s, w_rel, w_root, b, relu)
    out = _head(h, fc_w, fc_b)
    return out[:_N, :40]
